# baseline (speedup 1.0000x reference)
import math

import jax
import jax.numpy as jnp
from jax.experimental import pallas as pl
from jax.experimental.pallas import tpu as pltpu

BRANCH = 16
DIM = 2048
PROJ = 128
MAX_EXPERTS = 2
CLAMP_MAX = math.log(1.0 / 0.01)

TS = 256
CH = 256


def _dot_bf16x3(x, wcat):
    x_hi = x.astype(jnp.bfloat16)
    x_lo = (x - x_hi.astype(jnp.float32)).astype(jnp.bfloat16)
    xcat = jnp.concatenate([x_hi, x_hi, x_lo], axis=1)
    dn = (((1,), (0,)), ((), ()))
    return jax.lax.dot_general(xcat, wcat, dn,
                               preferred_element_type=jnp.float32)


def _post_block(y, b, g):
    y = y + b
    ss = jnp.sum(y * y, axis=1, keepdims=True)
    y = y * jax.lax.rsqrt(ss * (1.0 / PROJ) + 1e-6) * g
    y = 0.5 * y * (1.0 + jax.lax.erf(y * 0.7071067811865476))
    n2 = jnp.sum(y * y, axis=1, keepdims=True)
    y = y * jax.lax.rsqrt(jnp.maximum(n2, 1e-24))
    return jnp.sum(y, axis=0, keepdims=True)


def _gate_epilogue(p, sim, gates, temp, mask, logits_ref, topk_ref):
    pn = jnp.sum(p * p, axis=1, keepdims=True)
    fused = p * jax.lax.rsqrt(jnp.maximum(pn, 1e-24))
    cn = jnp.sum(sim * sim, axis=0, keepdims=True)
    simn = sim * jax.lax.rsqrt(jnp.maximum(cn, 1e-24))
    cos = jax.lax.dot_general(
        fused, simn, (((1,), (0,)), ((), ())),
        precision=jax.lax.Precision.HIGHEST,
        preferred_element_type=jnp.float32,
    )
    scale = jnp.exp(jnp.minimum(temp[0, 0], CLAMP_MAX))
    logits = jax.nn.sigmoid(cos * scale) * mask
    gsig = jax.nn.sigmoid(gates * scale)
    diff = logits - gsig

    sel = diff > 0.0
    cnt = jnp.sum(sel.astype(jnp.int32), axis=1, keepdims=True)
    iota = jax.lax.broadcasted_iota(jnp.int32, diff.shape, 1)
    neginf = jnp.float32(-jnp.inf)
    big = jnp.int32(10**6)

    m0 = jnp.max(diff, axis=1, keepdims=True)
    i0 = jnp.min(jnp.where(diff == m0, iota, big), axis=1, keepdims=True)
    keep_zero = iota == i0

    dm = jnp.where(sel, diff, neginf)
    m1 = jnp.max(dm, axis=1, keepdims=True)
    i1 = jnp.min(jnp.where(dm == m1, iota, big), axis=1, keepdims=True)
    is1 = iota == i1
    dm2 = jnp.where(is1, neginf, dm)
    m2 = jnp.max(dm2, axis=1, keepdims=True)
    i2 = jnp.min(jnp.where(dm2 == m2, iota, big), axis=1, keepdims=True)
    is2 = iota == i2
    keep_over = is1 | is2

    is_zero = (cnt == 0).astype(jnp.float32)
    is_over = (cnt > MAX_EXPERTS).astype(jnp.float32)
    selfl = sel.astype(jnp.float32)
    kzf = keep_zero.astype(jnp.float32)
    kof = keep_over.astype(jnp.float32)
    new = is_zero * kzf + (1.0 - is_zero) * (
        is_over * kof + (1.0 - is_over) * selfl)
    logits_ref[:] = new
    topk_ref[:] = jnp.clip(cnt, 1, MAX_EXPERTS)


def _make_body(nblocks, batch):
    bpb = nblocks // batch

    def body(x1_ref, x2_ref, w1_ref, b1_ref, g1_ref,
             w2_ref, b2_ref, g2_ref,
             sim_ref, gates_ref, temp_ref, mask_ref,
             logits_ref, topk_ref, acc_ref):
        i = pl.program_id(0)
        s = jnp.zeros((1, PROJ), jnp.float32)
        for c in range(TS // CH):
            sl = pl.ds(c * CH, CH)
            y = _dot_bf16x3(x1_ref[sl, :], w1_ref[:])
            s += _post_block(y, b1_ref[:], g1_ref[:])
        for c in range(TS // CH):
            sl = pl.ds(c * CH, CH)
            y = _dot_bf16x3(x2_ref[sl, :], w2_ref[:])
            s += _post_block(y, b2_ref[:], g2_ref[:])
        acc_ref[pl.ds(i, 1), :] = s

        @pl.when(i == nblocks - 1)
        def _():
            acc = acc_ref[:]
            p = jnp.sum(acc.reshape(batch, bpb, PROJ), axis=1)
            _gate_epilogue(p, sim_ref[:], gates_ref[:], temp_ref[:],
                           mask_ref[:], logits_ref, topk_ref)

    return body


@jax.jit
def kernel(x1, x2, W1, b1, g1, W2, b2, g2, sim_matrix, gates, temperature,
           experts_mask):
    B, S, _ = x1.shape
    nt = B * S
    nblocks = nt // TS
    xr1 = x1.reshape(nt, DIM)
    xr2 = x2.reshape(nt, DIM)
    w1t = W1.T
    w2t = W2.T
    w1h = w1t.astype(jnp.bfloat16)
    w1l = (w1t - w1h.astype(jnp.float32)).astype(jnp.bfloat16)
    w2h = w2t.astype(jnp.bfloat16)
    w2l = (w2t - w2h.astype(jnp.float32)).astype(jnp.bfloat16)
    w1cat = jnp.concatenate([w1h, w1l, w1h], axis=0)
    w2cat = jnp.concatenate([w2h, w2l, w2h], axis=0)

    row = lambda i: (0, 0)
    new_logits, topk = pl.pallas_call(
        _make_body(nblocks, B),
        grid=(nblocks,),
        in_specs=[
            pl.BlockSpec((TS, DIM), lambda i: (i, 0)),
            pl.BlockSpec((TS, DIM), lambda i: (i, 0)),
            pl.BlockSpec((3 * DIM, PROJ), row),
            pl.BlockSpec((1, PROJ), row),
            pl.BlockSpec((1, PROJ), row),
            pl.BlockSpec((3 * DIM, PROJ), row),
            pl.BlockSpec((1, PROJ), row),
            pl.BlockSpec((1, PROJ), row),
            pl.BlockSpec((PROJ, BRANCH), row),
            pl.BlockSpec((1, BRANCH), row),
            pl.BlockSpec((1, 1), row),
            pl.BlockSpec((1, BRANCH), row),
        ],
        out_specs=[
            pl.BlockSpec((B, BRANCH), row),
            pl.BlockSpec((B, 1), row),
        ],
        out_shape=[
            jax.ShapeDtypeStruct((B, BRANCH), jnp.float32),
            jax.ShapeDtypeStruct((B, 1), jnp.int32),
        ],
        scratch_shapes=[pltpu.VMEM((nblocks, PROJ), jnp.float32)],
    )(xr1, xr2, w1cat, b1.reshape(1, PROJ), g1.reshape(1, PROJ),
      w2cat, b2.reshape(1, PROJ), g2.reshape(1, PROJ),
      sim_matrix, gates.reshape(1, BRANCH), temperature.reshape(1, 1),
      experts_mask.reshape(1, BRANCH))

    return new_logits, topk.reshape(B).astype(jnp.int32)


# speedup vs baseline: 1.1730x; 1.1730x over previous
import math

import jax
import jax.numpy as jnp
from jax.experimental import pallas as pl
from jax.experimental.pallas import tpu as pltpu

BRANCH = 16
DIM = 2048
PROJ = 128
MAX_EXPERTS = 2
CLAMP_MAX = math.log(1.0 / 0.01)

TS = 1024
CH = 512


def _dot_bf16x3(x, wcat):
    x_hi = x.astype(jnp.bfloat16)
    x_lo = (x - x_hi.astype(jnp.float32)).astype(jnp.bfloat16)
    xcat = jnp.concatenate([x_hi, x_hi, x_lo], axis=1)
    dn = (((1,), (0,)), ((), ()))
    return jax.lax.dot_general(xcat, wcat, dn,
                               preferred_element_type=jnp.float32)


def _post_block(y, b, g):
    y = y + b
    ss = jnp.sum(y * y, axis=1, keepdims=True)
    y = y * jax.lax.rsqrt(ss * (1.0 / PROJ) + 1e-6) * g
    y = 0.5 * y * (1.0 + jax.lax.erf(y * 0.7071067811865476))
    n2 = jnp.sum(y * y, axis=1, keepdims=True)
    y = y * jax.lax.rsqrt(jnp.maximum(n2, 1e-24))
    return jnp.sum(y, axis=0, keepdims=True)


def _gate_epilogue(p, sim, gates, temp, mask, logits_ref, topk_ref):
    pn = jnp.sum(p * p, axis=1, keepdims=True)
    fused = p * jax.lax.rsqrt(jnp.maximum(pn, 1e-24))
    cn = jnp.sum(sim * sim, axis=0, keepdims=True)
    simn = sim * jax.lax.rsqrt(jnp.maximum(cn, 1e-24))
    cos = jax.lax.dot_general(
        fused, simn, (((1,), (0,)), ((), ())),
        precision=jax.lax.Precision.HIGHEST,
        preferred_element_type=jnp.float32,
    )
    scale = jnp.exp(jnp.minimum(temp[0, 0], CLAMP_MAX))
    logits = jax.nn.sigmoid(cos * scale) * mask
    gsig = jax.nn.sigmoid(gates * scale)
    diff = logits - gsig

    sel = diff > 0.0
    cnt = jnp.sum(sel.astype(jnp.int32), axis=1, keepdims=True)
    iota = jax.lax.broadcasted_iota(jnp.int32, diff.shape, 1)
    neginf = jnp.float32(-jnp.inf)
    big = jnp.int32(10**6)

    m0 = jnp.max(diff, axis=1, keepdims=True)
    i0 = jnp.min(jnp.where(diff == m0, iota, big), axis=1, keepdims=True)
    keep_zero = iota == i0

    dm = jnp.where(sel, diff, neginf)
    m1 = jnp.max(dm, axis=1, keepdims=True)
    i1 = jnp.min(jnp.where(dm == m1, iota, big), axis=1, keepdims=True)
    is1 = iota == i1
    dm2 = jnp.where(is1, neginf, dm)
    m2 = jnp.max(dm2, axis=1, keepdims=True)
    i2 = jnp.min(jnp.where(dm2 == m2, iota, big), axis=1, keepdims=True)
    is2 = iota == i2
    keep_over = is1 | is2

    is_zero = (cnt == 0).astype(jnp.float32)
    is_over = (cnt > MAX_EXPERTS).astype(jnp.float32)
    selfl = sel.astype(jnp.float32)
    kzf = keep_zero.astype(jnp.float32)
    kof = keep_over.astype(jnp.float32)
    new = is_zero * kzf + (1.0 - is_zero) * (
        is_over * kof + (1.0 - is_over) * selfl)
    logits_ref[:] = new
    topk_ref[:] = jnp.clip(cnt, 1, MAX_EXPERTS)


def _make_body(nblocks, batch):
    bpb = nblocks // batch

    def body(x1_ref, x2_ref, w1_ref, b1_ref, g1_ref,
             w2_ref, b2_ref, g2_ref,
             sim_ref, gates_ref, temp_ref, mask_ref,
             logits_ref, topk_ref, acc_ref):
        i = pl.program_id(0)
        s = jnp.zeros((1, PROJ), jnp.float32)
        for c in range(TS // CH):
            sl = pl.ds(c * CH, CH)
            y = _dot_bf16x3(x1_ref[sl, :], w1_ref[:])
            s += _post_block(y, b1_ref[:], g1_ref[:])
        for c in range(TS // CH):
            sl = pl.ds(c * CH, CH)
            y = _dot_bf16x3(x2_ref[sl, :], w2_ref[:])
            s += _post_block(y, b2_ref[:], g2_ref[:])
        acc_ref[pl.ds(i, 1), :] = s

        @pl.when(i == nblocks - 1)
        def _():
            acc = acc_ref[:]
            p = jnp.sum(acc.reshape(batch, bpb, PROJ), axis=1)
            _gate_epilogue(p, sim_ref[:], gates_ref[:], temp_ref[:],
                           mask_ref[:], logits_ref, topk_ref)

    return body


@jax.jit
def kernel(x1, x2, W1, b1, g1, W2, b2, g2, sim_matrix, gates, temperature,
           experts_mask):
    B, S, _ = x1.shape
    nt = B * S
    nblocks = nt // TS
    xr1 = x1.reshape(nt, DIM)
    xr2 = x2.reshape(nt, DIM)
    w1t = W1.T
    w2t = W2.T
    w1h = w1t.astype(jnp.bfloat16)
    w1l = (w1t - w1h.astype(jnp.float32)).astype(jnp.bfloat16)
    w2h = w2t.astype(jnp.bfloat16)
    w2l = (w2t - w2h.astype(jnp.float32)).astype(jnp.bfloat16)
    w1cat = jnp.concatenate([w1h, w1l, w1h], axis=0)
    w2cat = jnp.concatenate([w2h, w2l, w2h], axis=0)

    row = lambda i: (0, 0)
    new_logits, topk = pl.pallas_call(
        _make_body(nblocks, B),
        grid=(nblocks,),
        in_specs=[
            pl.BlockSpec((TS, DIM), lambda i: (i, 0)),
            pl.BlockSpec((TS, DIM), lambda i: (i, 0)),
            pl.BlockSpec((3 * DIM, PROJ), row),
            pl.BlockSpec((1, PROJ), row),
            pl.BlockSpec((1, PROJ), row),
            pl.BlockSpec((3 * DIM, PROJ), row),
            pl.BlockSpec((1, PROJ), row),
            pl.BlockSpec((1, PROJ), row),
            pl.BlockSpec((PROJ, BRANCH), row),
            pl.BlockSpec((1, BRANCH), row),
            pl.BlockSpec((1, 1), row),
            pl.BlockSpec((1, BRANCH), row),
        ],
        out_specs=[
            pl.BlockSpec((B, BRANCH), row),
            pl.BlockSpec((B, 1), row),
        ],
        out_shape=[
            jax.ShapeDtypeStruct((B, BRANCH), jnp.float32),
            jax.ShapeDtypeStruct((B, 1), jnp.int32),
        ],
        scratch_shapes=[pltpu.VMEM((nblocks, PROJ), jnp.float32)],
    )(xr1, xr2, w1cat, b1.reshape(1, PROJ), g1.reshape(1, PROJ),
      w2cat, b2.reshape(1, PROJ), g2.reshape(1, PROJ),
      sim_matrix, gates.reshape(1, BRANCH), temperature.reshape(1, 1),
      experts_mask.reshape(1, BRANCH))

    return new_logits, topk.reshape(B).astype(jnp.int32)
